# Initial kernel scaffold; baseline (speedup 1.0000x reference)
import jax
import jax.numpy as jnp
from jax.experimental import pallas as pl


def kernel(x, edge_index, W1, a_src1, a_dst1, b1, W2, a_src2, a_dst2, b2):
    raise NotImplementedError("write your pallas kernel here")


# speedup vs baseline: 1.0001x; 1.0001x over previous
import jax
import jax.numpy as jnp
from jax.experimental import pallas as pl

N = 50000
HID = 64
HEADS = 4
NC_OUT = 16


def _edge_softmax_aggregate(x_feat, src, dst, alpha_s, alpha_d, n):
    e = alpha_s[src] + alpha_d[dst]
    e = jnp.where(e > 0, e, 0.2 * e)
    m = jax.ops.segment_max(e, dst, num_segments=n)
    ex = jnp.exp(e - m[dst])
    den = jax.ops.segment_sum(ex, dst, num_segments=n)
    alpha = ex / (den[dst] + 1e-16)
    return jax.ops.segment_sum(x_feat[src] * alpha[..., None], dst, num_segments=n)


def _logsoftmax_body(x_ref, o_ref):
    x = x_ref[...]
    m = jnp.max(x, axis=1, keepdims=True)
    s = jnp.log(jnp.sum(jnp.exp(x - m), axis=1, keepdims=True))
    o_ref[...] = x - m - s


def kernel(x, edge_index, W1, a_src1, a_dst1, b1, W2, a_src2, a_dst2, b2):
    n = x.shape[0]
    loop = jnp.arange(n, dtype=edge_index.dtype)
    src = jnp.concatenate([edge_index[0], loop])
    dst = jnp.concatenate([edge_index[1], loop])

    h1 = (x @ W1).reshape(n, HEADS, HID)
    as1 = jnp.sum(h1 * a_src1[None], axis=-1)
    ad1 = jnp.sum(h1 * a_dst1[None], axis=-1)
    out1 = _edge_softmax_aggregate(h1, src, dst, as1, ad1, n)
    out1 = out1.reshape(n, HEADS * HID) + b1
    h = jax.nn.elu(out1)

    h2 = h @ W2
    as2 = jnp.sum(h2 * a_src2, axis=-1)
    ad2 = jnp.sum(h2 * a_dst2, axis=-1)
    out2 = _edge_softmax_aggregate(h2, src, dst, as2, ad2, n)
    out2 = out2 + b2

    blk = 2000
    return pl.pallas_call(
        _logsoftmax_body,
        grid=(n // blk,),
        in_specs=[pl.BlockSpec((blk, NC_OUT), lambda i: (i, 0))],
        out_specs=pl.BlockSpec((blk, NC_OUT), lambda i: (i, 0)),
        out_shape=jax.ShapeDtypeStruct((n, NC_OUT), jnp.float32),
    )(out2)


# speedup vs baseline: 13.2390x; 13.2375x over previous
import jax
import jax.numpy as jnp
from jax import lax
from jax.experimental import pallas as pl
from jax.experimental.pallas import tpu as pltpu
from jax.experimental.pallas import tpu_sc as plsc

N = 50000
E = 800000
HID = 64
HEADS = 4
NCO = 16

NWK = 32
EC = 25088
EPAD = NWK * EC
NBK = 7
CAP = 5120
NCOV = NBK * 8192
R1 = 256
NPASS = 7
PK = 272
GROW = 32
XW = HID + 16
WG = 128
CH = 512
CCAP = 6272

_SC_PARAMS = pltpu.CompilerParams(
    needs_layout_passes=False, use_tc_tiling_on_sc=False)


def _lrelu(v):
    return jnp.maximum(v, 0.2 * v)


def _widx():
    return lax.axis_index("s") * 2 + lax.axis_index("c")


def _tc_a_body(x_ref, v1s_ref, v1d_ref, xpk_ref, ad1_ref):
    xb = x_ref[...]
    s = jnp.dot(xb, v1s_ref[...], preferred_element_type=jnp.float32)
    d = jnp.dot(xb, v1d_ref[...], preferred_element_type=jnp.float32)
    pad = jnp.zeros((xb.shape[0], XW - HID - HEADS), jnp.float32)
    xpk_ref[...] = jnp.concatenate([xb, s, pad], axis=1)
    ad1_ref[...] = d


def _sc_bin_body(srcp, dstp, bsrc, bdst, counts,
                 chs, chd, cs, cd, cntb):
    wid = _widx()
    lane = lax.iota(jnp.int32, 16)
    pltpu.sync_copy(srcp.at[pl.ds(wid * EC, EC)], chs)
    pltpu.sync_copy(dstp.at[pl.ds(wid * EC, EC)], chd)

    def zro(i, _):
        cntb[pl.ds(i * 16, 16)] = jnp.zeros((16,), jnp.int32)
        return 0

    lax.fori_loop(0, 8, zro, 0)

    def bkt(b, _):
        def grp(g, cur):
            s16 = chs[pl.ds(g * 16, 16)]
            d16 = chd[pl.ds(g * 16, 16)]
            m = lax.shift_right_logical(d16, 13) == b
            pc = plsc.cumsum(m.astype(jnp.int32))
            idx = cur + pc - 1
            plsc.store_scatter(cs, [idx], s16, mask=m)
            plsc.store_scatter(cd, [idx], d16, mask=m)
            return cur + jnp.max(pc)

        cur = lax.fori_loop(0, EC // 16, grp, 0)
        plsc.store_scatter(cntb, [jnp.full((16,), b * 8, jnp.int32)],
                           jnp.full((16,), cur, jnp.int32))
        pltpu.sync_copy(cs, bsrc.at[pl.ds((wid * NBK + b) * CAP, CAP)])
        pltpu.sync_copy(cd, bdst.at[pl.ds((wid * NBK + b) * CAP, CAP)])
        return 0

    lax.fori_loop(0, NBK, bkt, 0)
    pltpu.sync_copy(cntb, counts.at[pl.ds(wid * 128, 128)])


def _compact_pass(p, mylo, rng, cntb, bsrc, bdst, sbuf, dbuf, cs, cd):
    lane = lax.iota(jnp.int32, 16)

    def src_tile(t2, cur):
        cnt = cntb[pl.ds(t2 * 128 + p * 8, 16)][0]
        lbase = (t2 * NBK + p) * CAP
        nch = (cnt + CH - 1) // CH

        def chunk(ch, cur):
            pltpu.sync_copy(bsrc.at[pl.ds(lbase + ch * CH, CH)], sbuf)
            pltpu.sync_copy(bdst.at[pl.ds(lbase + ch * CH, CH)], dbuf)

            def grp(g, cur):
                pos = ch * CH + g * 16 + lane
                s16 = sbuf[pl.ds(g * 16, 16)]
                d16 = dbuf[pl.ds(g * 16, 16)]
                dl = d16 - mylo
                m = (pos < cnt) & (d16 >= mylo) & (d16 < mylo + rng)
                pc = plsc.cumsum(m.astype(jnp.int32))
                idx = cur + pc - 1
                plsc.store_scatter(cs, [idx], s16, mask=m)
                plsc.store_scatter(cd, [idx], dl, mask=m)
                return cur + jnp.max(pc)

            return lax.fori_loop(0, CH // 16, grp, cur)

        return lax.fori_loop(0, nch, chunk, cur)

    return lax.fori_loop(0, NWK, src_tile, 0)


def _sc1_body(xpk, bsrc, bdst, counts, ad1p, z1, p1,
              acc, ad1t, cntb, sbuf, dbuf, cs, cd, csw, cdw, gbuf, wbuf,
              sem):
    wid = _widx()
    lane = lax.iota(jnp.int32, 16)
    pltpu.sync_copy(counts, cntb)

    def ini(i, _):
        z = jnp.zeros((16,), jnp.int32)
        cs[pl.ds(i * 16, 16)] = z
        cd[pl.ds(i * 16, 16)] = z
        wbuf[pl.ds(i * 16, 16)] = jnp.zeros((16,), jnp.float32)
        return 0

    lax.fori_loop(0, CCAP // 16, ini, 0)

    def do_pass(p, _):
        mylo = p * 8192 + wid * R1
        pltpu.sync_copy(z1, acc)
        pltpu.sync_copy(ad1p.at[pl.ds(mylo, R1)], ad1t)
        k_cnt = _compact_pass(p, mylo, R1, cntb, bsrc, bdst,
                              sbuf, dbuf, cs, cd)
        n_sub = (k_cnt + WG - 1) // WG

        def subw(j, _):
            for q in range(WG // 16):
                csw[pl.ds(q * 16, 16)] = cs[pl.ds(j * WG + q * 16, 16)]
                cdw[pl.ds(q * 16, 16)] = cd[pl.ds(j * WG + q * 16, 16)]
            pltpu.async_copy(xpk.at[csw], gbuf, sem).wait()

            def wgrp(q, _):
                row = q * 16 + lane
                dl = cdw[pl.ds(q * 16, 16)]
                live = (j * WG + row) < k_cnt
                for h in range(HEADS):
                    a_s = plsc.load_gather(
                        gbuf, [row, jnp.full((16,), HID + h, jnp.int32)])
                    a_d = plsc.load_gather(
                        ad1t, [dl, jnp.full((16,), h, jnp.int32)])
                    wv = jnp.exp(_lrelu(a_s + a_d))
                    wv = jnp.where(live, wv, 0.0)
                    plsc.store_scatter(wbuf, [row * 16 + h], wv)
                return 0

            lax.fori_loop(0, WG // 16, wgrp, 0)

            def egrp(g, _):
                dl16 = cdw[pl.ds(g * 16, 16)]
                for kk in range(16):
                    dl = dl16[kk]
                    row = g * 16 + kk
                    wrow = wbuf[pl.ds(row * 16, 16)]
                    for h in range(HEADS):
                        wsp = jnp.full((16,), wrow[h])
                        for jx in range(HID // 16):
                            xv = gbuf[row, pl.ds(jx * 16, 16)]
                            col = h * HID + jx * 16
                            cv = acc[dl, pl.ds(col, 16)]
                            acc[dl, pl.ds(col, 16)] = cv + wsp * xv
                    dv = acc[dl, pl.ds(4 * HID, 16)]
                    acc[dl, pl.ds(4 * HID, 16)] = dv + wrow
                return 0

            lax.fori_loop(0, WG // 16, egrp, 0)
            return 0

        lax.fori_loop(0, n_sub, subw, 0)
        pltpu.sync_copy(acc, p1.at[pl.ds(mylo, R1)])
        return 0

    lax.fori_loop(0, NPASS, do_pass, 0)


def _tc_b_body(p1_ref, xpk_ref, ad1_ref, w1_ref, b1_ref, w2_ref,
               as2t_ref, ad2t_ref, h2pk_ref, ad2row_ref):
    ps = p1_ref[...]
    xb = xpk_ref[:, :HID]
    as1 = xpk_ref[:, HID:HID + HEADS]
    ad1 = ad1_ref[...]
    sw = jnp.exp(_lrelu(as1 + ad1))
    outs = []
    for h in range(HEADS):
        ah = ps[:, h * HID:(h + 1) * HID] + sw[:, h:h + 1] * xb
        dh = ps[:, 4 * HID + h:4 * HID + h + 1] + sw[:, h:h + 1] + 1e-16
        outs.append(jnp.dot(ah / dh, w1_ref[:, h * HID:(h + 1) * HID],
                            preferred_element_type=jnp.float32))
    o1 = jnp.concatenate(outs, axis=1) + b1_ref[...]
    he = jnp.where(o1 > 0, o1, jnp.exp(jnp.minimum(o1, 0.0)) - 1.0)
    h2 = jnp.dot(he, w2_ref[...], preferred_element_type=jnp.float32)
    s2 = jnp.dot(h2, as2t_ref[...], preferred_element_type=jnp.float32)
    d2 = jnp.dot(h2, ad2t_ref[...], preferred_element_type=jnp.float32)
    pad = jnp.zeros((h2.shape[0], GROW - NCO - 2), jnp.float32)
    h2pk_ref[...] = jnp.concatenate([h2, s2, d2, pad], axis=1)
    ad2row_ref[...] = jnp.reshape(
        lax.dot_general(jnp.transpose(ad2t_ref[...]), h2,
                        (((1,), (1,)), ((), ())),
                        preferred_element_type=jnp.float32),
        (1, 1, h2.shape[0]))


def _sc2_body(h2pk, bsrc, bdst, counts, ad2p, z2, p2,
              acc, ad2t, cntb, sbuf, dbuf, cs, cd, csw, cdw, gbuf, wbuf,
              sem):
    wid = _widx()
    lane = lax.iota(jnp.int32, 16)
    pltpu.sync_copy(counts, cntb)

    def ini(i, _):
        z = jnp.zeros((16,), jnp.int32)
        cs[pl.ds(i * 16, 16)] = z
        cd[pl.ds(i * 16, 16)] = z
        wbuf[pl.ds(i * 16, 16)] = jnp.zeros((16,), jnp.float32)
        return 0

    lax.fori_loop(0, CCAP // 16, ini, 0)

    def do_pass(p, _):
        mylo = p * 8192 + wid * R1
        pltpu.sync_copy(z2, acc)
        pltpu.sync_copy(ad2p.at[pl.ds(mylo, R1)], ad2t)
        k_cnt = _compact_pass(p, mylo, R1, cntb, bsrc, bdst,
                              sbuf, dbuf, cs, cd)
        n_sub = (k_cnt + WG - 1) // WG

        def subw(j, _):
            for q in range(WG // 16):
                csw[pl.ds(q * 16, 16)] = cs[pl.ds(j * WG + q * 16, 16)]
                cdw[pl.ds(q * 16, 16)] = cd[pl.ds(j * WG + q * 16, 16)]
            pltpu.async_copy(h2pk.at[csw], gbuf, sem).wait()

            def wgrp(q, _):
                row = q * 16 + lane
                dl = cdw[pl.ds(q * 16, 16)]
                live = (j * WG + row) < k_cnt
                a_s = plsc.load_gather(
                    gbuf, [row, jnp.full((16,), NCO, jnp.int32)])
                a_d = plsc.load_gather(ad2t, [dl])
                wv = jnp.exp(_lrelu(a_s + a_d))
                wv = jnp.where(live, wv, 0.0)
                plsc.store_scatter(wbuf, [row * 16], wv)
                return 0

            lax.fori_loop(0, WG // 16, wgrp, 0)

            def egrp(g, _):
                dl16 = cdw[pl.ds(g * 16, 16)]
                for kk in range(16):
                    dl = dl16[kk]
                    row = g * 16 + kk
                    wrow = wbuf[pl.ds(row * 16, 16)]
                    wsp = jnp.full((16,), wrow[0])
                    xv = gbuf[row, pl.ds(0, 16)]
                    cv = acc[dl, pl.ds(0, 16)]
                    acc[dl, pl.ds(0, 16)] = cv + wsp * xv
                    dv = acc[dl, pl.ds(16, 16)]
                    acc[dl, pl.ds(16, 16)] = dv + wrow
                return 0

            lax.fori_loop(0, WG // 16, egrp, 0)
            return 0

        lax.fori_loop(0, n_sub, subw, 0)
        pltpu.sync_copy(acc, p2.at[pl.ds(mylo, R1)])
        return 0

    lax.fori_loop(0, NPASS, do_pass, 0)


def _tc_c_body(p2_ref, h2pk_ref, b2_ref, out_ref):
    pb = p2_ref[...]
    aggs = pb[:, :NCO]
    dens = pb[:, NCO:NCO + 1]
    hb = h2pk_ref[...]
    h2 = hb[:, :NCO]
    s2 = hb[:, NCO:NCO + 1]
    d2 = hb[:, NCO + 1:NCO + 2]
    sw = jnp.exp(_lrelu(s2 + d2))
    num = aggs + sw * h2
    den = dens + sw + 1e-16
    o = num / den + b2_ref[...]
    m = jnp.max(o, axis=1, keepdims=True)
    ls = jnp.log(jnp.sum(jnp.exp(o - m), axis=1, keepdims=True))
    out_ref[...] = o - m - ls


def kernel(x, edge_index, W1, a_src1, a_dst1, b1, W2, a_src2, a_dst2, b2):
    f32 = jnp.float32
    i32 = jnp.int32
    blk = 2000
    grid = (N // blk,)
    scmesh = plsc.VectorSubcoreMesh(core_axis_name="c", subcore_axis_name="s")

    w1r = W1.reshape(HID, HEADS, HID)
    v1s = jnp.einsum("khc,hc->kh", w1r, a_src1)
    v1d = jnp.einsum("khc,hc->kh", w1r, a_dst1)
    npd = EPAD - E
    psrc = (jnp.arange(npd, dtype=i32) * 97) % N
    pdst = N + 48 + jnp.arange(npd, dtype=i32) % 128
    srcp = jnp.concatenate([edge_index[0], psrc])
    dstp = jnp.concatenate([edge_index[1], pdst])
    z1 = jnp.zeros((R1, PK), f32)
    z2 = jnp.zeros((R1, GROW), f32)
    b1r = b1.reshape(1, HEADS * HID)
    b2r = b2.reshape(1, NCO)
    as2t = a_src2.reshape(NCO, 1)
    ad2t = a_dst2.reshape(NCO, 1)

    xpk, ad1 = pl.pallas_call(
        _tc_a_body,
        grid=grid,
        in_specs=[
            pl.BlockSpec((blk, HID), lambda i: (i, 0)),
            pl.BlockSpec((HID, HEADS), lambda i: (0, 0)),
            pl.BlockSpec((HID, HEADS), lambda i: (0, 0)),
        ],
        out_specs=[
            pl.BlockSpec((blk, XW), lambda i: (i, 0)),
            pl.BlockSpec((blk, HEADS), lambda i: (i, 0)),
        ],
        out_shape=[
            jax.ShapeDtypeStruct((N, XW), f32),
            jax.ShapeDtypeStruct((N, HEADS), f32),
        ],
    )(x, v1s, v1d)
    ad1p = jnp.concatenate([ad1, jnp.zeros((NCOV - N, HEADS), f32)])

    sc_bin = pl.kernel(
        _sc_bin_body,
        out_type=[
            jax.ShapeDtypeStruct((NWK * NBK * CAP,), i32),
            jax.ShapeDtypeStruct((NWK * NBK * CAP,), i32),
            jax.ShapeDtypeStruct((NWK * 128,), i32),
        ],
        mesh=scmesh,
        compiler_params=_SC_PARAMS,
        scratch_types=[
            pltpu.VMEM((EC,), i32),
            pltpu.VMEM((EC,), i32),
            pltpu.VMEM((CAP,), i32),
            pltpu.VMEM((CAP,), i32),
            pltpu.VMEM((128,), i32),
        ],
    )
    bsrc, bdst, counts = sc_bin(srcp, dstp)

    sc1 = pl.kernel(
        _sc1_body,
        out_type=[jax.ShapeDtypeStruct((NCOV, PK), f32)],
        mesh=scmesh,
        compiler_params=_SC_PARAMS,
        scratch_types=[
            pltpu.VMEM((R1, PK), f32),
            pltpu.VMEM((R1, HEADS), f32),
            pltpu.VMEM((NWK * 128,), i32),
            pltpu.VMEM((CH,), i32),
            pltpu.VMEM((CH,), i32),
            pltpu.VMEM((CCAP,), i32),
            pltpu.VMEM((CCAP,), i32),
            pltpu.VMEM((WG,), i32),
            pltpu.VMEM((WG,), i32),
            pltpu.VMEM((WG, XW), f32),
            pltpu.VMEM((WG * 16,), f32),
            pltpu.SemaphoreType.DMA,
        ],
    )
    (p1,) = sc1(xpk, bsrc, bdst, counts, ad1p, z1)

    h2pk, ad2row = pl.pallas_call(
        _tc_b_body,
        grid=grid,
        in_specs=[
            pl.BlockSpec((blk, PK), lambda i: (i, 0)),
            pl.BlockSpec((blk, XW), lambda i: (i, 0)),
            pl.BlockSpec((blk, HEADS), lambda i: (i, 0)),
            pl.BlockSpec((HID, HEADS * HID), lambda i: (0, 0)),
            pl.BlockSpec((1, HEADS * HID), lambda i: (0, 0)),
            pl.BlockSpec((HEADS * HID, NCO), lambda i: (0, 0)),
            pl.BlockSpec((NCO, 1), lambda i: (0, 0)),
            pl.BlockSpec((NCO, 1), lambda i: (0, 0)),
        ],
        out_specs=[
            pl.BlockSpec((blk, GROW), lambda i: (i, 0)),
            pl.BlockSpec((1, 1, blk), lambda i: (i, 0, 0)),
        ],
        out_shape=[
            jax.ShapeDtypeStruct((N, GROW), f32),
            jax.ShapeDtypeStruct((N // blk, 1, blk), f32),
        ],
    )(p1, xpk, ad1, W1, b1r, W2, as2t, ad2t)
    ad2p = jnp.concatenate([ad2row.reshape(N), jnp.zeros((NCOV - N,), f32)])

    sc2 = pl.kernel(
        _sc2_body,
        out_type=[jax.ShapeDtypeStruct((NCOV, GROW), f32)],
        mesh=scmesh,
        compiler_params=_SC_PARAMS,
        scratch_types=[
            pltpu.VMEM((R1, GROW), f32),
            pltpu.VMEM((R1,), f32),
            pltpu.VMEM((NWK * 128,), i32),
            pltpu.VMEM((CH,), i32),
            pltpu.VMEM((CH,), i32),
            pltpu.VMEM((CCAP,), i32),
            pltpu.VMEM((CCAP,), i32),
            pltpu.VMEM((WG,), i32),
            pltpu.VMEM((WG,), i32),
            pltpu.VMEM((WG, GROW), f32),
            pltpu.VMEM((WG * 16,), f32),
            pltpu.SemaphoreType.DMA,
        ],
    )
    (p2,) = sc2(h2pk, bsrc, bdst, counts, ad2p, z2)

    return pl.pallas_call(
        _tc_c_body,
        grid=grid,
        in_specs=[
            pl.BlockSpec((blk, GROW), lambda i: (i, 0)),
            pl.BlockSpec((blk, GROW), lambda i: (i, 0)),
            pl.BlockSpec((1, NCO), lambda i: (0, 0)),
        ],
        out_specs=pl.BlockSpec((blk, NCO), lambda i: (i, 0)),
        out_shape=jax.ShapeDtypeStruct((N, NCO), f32),
    )(p2, h2pk, b2r)


# speedup vs baseline: 23.5335x; 1.7776x over previous
import jax
import jax.numpy as jnp
from jax import lax
from jax.experimental import pallas as pl
from jax.experimental.pallas import tpu as pltpu
from jax.experimental.pallas import tpu_sc as plsc

N = 50000
E = 800000
HID = 64
HEADS = 4
NCO = 16

NWK = 32
EC = 25088
EPAD = NWK * EC
NBK = 7
CAP = 6144
NCOV = NBK * 8192
R1 = 256
NPASS = 7
PK = 272
GROW = 32
XW = HID + 16
WG = 128
CH = 2048
CCAP = 6272

_SC_PARAMS = pltpu.CompilerParams(
    needs_layout_passes=False, use_tc_tiling_on_sc=False)


def _lrelu(v):
    return jnp.maximum(v, 0.2 * v)


def _widx():
    return lax.axis_index("s") * 2 + lax.axis_index("c")


def _tc_a_body(x_ref, v1s_ref, v1d_ref, xpk_ref, ad1_ref):
    xb = x_ref[...]
    s = jnp.dot(xb, v1s_ref[...], preferred_element_type=jnp.float32)
    d = jnp.dot(xb, v1d_ref[...], preferred_element_type=jnp.float32)
    pad = jnp.zeros((xb.shape[0], XW - HID - HEADS), jnp.float32)
    xpk_ref[...] = jnp.concatenate([xb, s, pad], axis=1)
    ad1_ref[...] = d


def _sc_bin_body(srcp, dstp, bsrc, bdst, counts,
                 chs, chd, cs, cd, cntb):
    wid = _widx()
    lane = lax.iota(jnp.int32, 16)
    pltpu.sync_copy(srcp.at[pl.ds(wid * EC, EC)], chs)
    pltpu.sync_copy(dstp.at[pl.ds(wid * EC, EC)], chd)

    def zro(i, _):
        cntb[pl.ds(i * 16, 16)] = jnp.zeros((16,), jnp.int32)
        return 0

    lax.fori_loop(0, 8, zro, 0)

    def bkt(b, _):
        def grp(g, cur):
            s16 = chs[pl.ds(g * 16, 16)]
            d16 = chd[pl.ds(g * 16, 16)]
            m = lax.shift_right_logical(d16, 13) == b
            pc = plsc.cumsum(m.astype(jnp.int32))
            idx = cur + pc - 1
            plsc.store_scatter(cs, [idx], s16, mask=m)
            plsc.store_scatter(cd, [idx], d16, mask=m)
            return cur + jnp.max(pc)

        cur = lax.fori_loop(0, EC // 16, grp, 0)
        plsc.store_scatter(cntb, [jnp.full((16,), b * 8, jnp.int32)],
                           jnp.full((16,), cur, jnp.int32))
        pltpu.sync_copy(cs, bsrc.at[pl.ds((wid * NBK + b) * CAP, CAP)])
        pltpu.sync_copy(cd, bdst.at[pl.ds((wid * NBK + b) * CAP, CAP)])
        return 0

    lax.fori_loop(0, NBK, bkt, 0)
    pltpu.sync_copy(cntb, counts.at[pl.ds(wid * 128, 128)])


def _compact_pass(p, mylo, rng, cntb, bsrc, bdst, sbuf, dbuf, cs, cd):
    lane = lax.iota(jnp.int32, 16)

    def src_tile(t2, cur):
        cnt = cntb[pl.ds(t2 * 128 + p * 8, 16)][0]
        lbase = (t2 * NBK + p) * CAP
        nch = (cnt + CH - 1) // CH

        def chunk(ch, cur):
            pltpu.sync_copy(bsrc.at[pl.ds(lbase + ch * CH, CH)], sbuf)
            pltpu.sync_copy(bdst.at[pl.ds(lbase + ch * CH, CH)], dbuf)

            def grp(g, cur):
                pos = ch * CH + g * 16 + lane
                s16 = sbuf[pl.ds(g * 16, 16)]
                d16 = dbuf[pl.ds(g * 16, 16)]
                dl = d16 - mylo
                m = (pos < cnt) & (d16 >= mylo) & (d16 < mylo + rng)
                pc = plsc.cumsum(m.astype(jnp.int32))
                idx = cur + pc - 1
                plsc.store_scatter(cs, [idx], s16, mask=m)
                plsc.store_scatter(cd, [idx], dl, mask=m)
                return cur + jnp.max(pc)

            return lax.fori_loop(0, CH // 16, grp, cur)

        return lax.fori_loop(0, nch, chunk, cur)

    return lax.fori_loop(0, NWK, src_tile, 0)


def _sc1_body(xpk, bsrc, bdst, counts, ad1p, z1, p1,
              acc, ad1t, cntb, sbuf, dbuf, cs, cd, csw, cdw, gbuf, wbuf,
              sem):
    wid = _widx()
    lane = lax.iota(jnp.int32, 16)
    pltpu.sync_copy(counts, cntb)

    def ini(i, _):
        z = jnp.zeros((16,), jnp.int32)
        cs[pl.ds(i * 16, 16)] = z
        cd[pl.ds(i * 16, 16)] = z
        wbuf[pl.ds(i * 16, 16)] = jnp.zeros((16,), jnp.float32)
        return 0

    lax.fori_loop(0, CCAP // 16, ini, 0)

    def do_pass(p, _):
        mylo = p * 8192 + wid * R1
        pltpu.sync_copy(z1, acc)
        pltpu.sync_copy(ad1p.at[pl.ds(mylo, R1)], ad1t)
        k_cnt = _compact_pass(p, mylo, R1, cntb, bsrc, bdst,
                              sbuf, dbuf, cs, cd)
        n_sub = (k_cnt + WG - 1) // WG

        def subw(j, _):
            for q in range(WG // 16):
                csw[pl.ds(q * 16, 16)] = cs[pl.ds(j * WG + q * 16, 16)]
                cdw[pl.ds(q * 16, 16)] = cd[pl.ds(j * WG + q * 16, 16)]
            pltpu.async_copy(xpk.at[csw], gbuf, sem).wait()

            def wgrp(q, _):
                row = q * 16 + lane
                dl = cdw[pl.ds(q * 16, 16)]
                live = (j * WG + row) < k_cnt
                for h in range(HEADS):
                    a_s = plsc.load_gather(
                        gbuf, [row, jnp.full((16,), HID + h, jnp.int32)])
                    a_d = plsc.load_gather(
                        ad1t, [dl, jnp.full((16,), h, jnp.int32)])
                    wv = jnp.exp(_lrelu(a_s + a_d))
                    wv = jnp.where(live, wv, 0.0)
                    plsc.store_scatter(wbuf, [row * 16 + h], wv)
                return 0

            lax.fori_loop(0, WG // 16, wgrp, 0)

            def egrp(g, _):
                dl16 = cdw[pl.ds(g * 16, 16)]
                for kk in range(16):
                    dl = dl16[kk]
                    row = g * 16 + kk
                    wrow = wbuf[pl.ds(row * 16, 16)]
                    wsps = [jnp.full((16,), wrow[h]) for h in range(HEADS)]
                    for jx in range(HID // 16):
                        xv = gbuf[row, pl.ds(jx * 16, 16)]
                        for h in range(HEADS):
                            col = h * HID + jx * 16
                            plsc.addupdate(acc.at[dl, pl.ds(col, 16)],
                                           wsps[h] * xv)
                    plsc.addupdate(acc.at[dl, pl.ds(4 * HID, 16)], wrow)
                return 0

            lax.fori_loop(0, WG // 16, egrp, 0)
            return 0

        lax.fori_loop(0, n_sub, subw, 0)
        pltpu.sync_copy(acc, p1.at[pl.ds(mylo, R1)])
        return 0

    lax.fori_loop(0, NPASS, do_pass, 0)


def _tc_b_body(p1_ref, xpk_ref, ad1_ref, w1_ref, b1_ref, w2_ref,
               as2t_ref, ad2t_ref, h2pk_ref, ad2row_ref):
    ps = p1_ref[...]
    xb = xpk_ref[:, :HID]
    as1 = xpk_ref[:, HID:HID + HEADS]
    ad1 = ad1_ref[...]
    sw = jnp.exp(_lrelu(as1 + ad1))
    outs = []
    for h in range(HEADS):
        ah = ps[:, h * HID:(h + 1) * HID] + sw[:, h:h + 1] * xb
        dh = ps[:, 4 * HID + h:4 * HID + h + 1] + sw[:, h:h + 1] + 1e-16
        outs.append(jnp.dot(ah / dh, w1_ref[:, h * HID:(h + 1) * HID],
                            preferred_element_type=jnp.float32))
    o1 = jnp.concatenate(outs, axis=1) + b1_ref[...]
    he = jnp.where(o1 > 0, o1, jnp.exp(jnp.minimum(o1, 0.0)) - 1.0)
    h2 = jnp.dot(he, w2_ref[...], preferred_element_type=jnp.float32)
    s2 = jnp.dot(h2, as2t_ref[...], preferred_element_type=jnp.float32)
    d2 = jnp.dot(h2, ad2t_ref[...], preferred_element_type=jnp.float32)
    pad = jnp.zeros((h2.shape[0], GROW - NCO - 2), jnp.float32)
    h2pk_ref[...] = jnp.concatenate([h2, s2, d2, pad], axis=1)
    ad2row_ref[...] = jnp.reshape(
        lax.dot_general(jnp.transpose(ad2t_ref[...]), h2,
                        (((1,), (1,)), ((), ())),
                        preferred_element_type=jnp.float32),
        (1, 1, h2.shape[0]))


def _sc2_body(h2pk, bsrc, bdst, counts, ad2p, z2, p2,
              acc, ad2t, cntb, sbuf, dbuf, cs, cd, csw, cdw, gbuf, wbuf,
              sem):
    wid = _widx()
    lane = lax.iota(jnp.int32, 16)
    pltpu.sync_copy(counts, cntb)

    def ini(i, _):
        z = jnp.zeros((16,), jnp.int32)
        cs[pl.ds(i * 16, 16)] = z
        cd[pl.ds(i * 16, 16)] = z
        wbuf[pl.ds(i * 16, 16)] = jnp.zeros((16,), jnp.float32)
        return 0

    lax.fori_loop(0, CCAP // 16, ini, 0)

    def do_pass(p, _):
        mylo = p * 8192 + wid * R1
        pltpu.sync_copy(z2, acc)
        pltpu.sync_copy(ad2p.at[pl.ds(mylo, R1)], ad2t)
        k_cnt = _compact_pass(p, mylo, R1, cntb, bsrc, bdst,
                              sbuf, dbuf, cs, cd)
        n_sub = (k_cnt + WG - 1) // WG

        def subw(j, _):
            for q in range(WG // 16):
                csw[pl.ds(q * 16, 16)] = cs[pl.ds(j * WG + q * 16, 16)]
                cdw[pl.ds(q * 16, 16)] = cd[pl.ds(j * WG + q * 16, 16)]
            pltpu.async_copy(h2pk.at[csw], gbuf, sem).wait()

            def wgrp(q, _):
                row = q * 16 + lane
                dl = cdw[pl.ds(q * 16, 16)]
                live = (j * WG + row) < k_cnt
                a_s = plsc.load_gather(
                    gbuf, [row, jnp.full((16,), NCO, jnp.int32)])
                a_d = plsc.load_gather(ad2t, [dl])
                wv = jnp.exp(_lrelu(a_s + a_d))
                wv = jnp.where(live, wv, 0.0)
                plsc.store_scatter(wbuf, [row * 16], wv)
                return 0

            lax.fori_loop(0, WG // 16, wgrp, 0)

            def egrp(g, _):
                dl16 = cdw[pl.ds(g * 16, 16)]
                for kk in range(16):
                    dl = dl16[kk]
                    row = g * 16 + kk
                    wrow = wbuf[pl.ds(row * 16, 16)]
                    wsp = jnp.full((16,), wrow[0])
                    xv = gbuf[row, pl.ds(0, 16)]
                    plsc.addupdate(acc.at[dl, pl.ds(0, 16)], wsp * xv)
                    plsc.addupdate(acc.at[dl, pl.ds(16, 16)], wrow)
                return 0

            lax.fori_loop(0, WG // 16, egrp, 0)
            return 0

        lax.fori_loop(0, n_sub, subw, 0)
        pltpu.sync_copy(acc, p2.at[pl.ds(mylo, R1)])
        return 0

    lax.fori_loop(0, NPASS, do_pass, 0)


def _tc_c_body(p2_ref, h2pk_ref, b2_ref, out_ref):
    pb = p2_ref[...]
    aggs = pb[:, :NCO]
    dens = pb[:, NCO:NCO + 1]
    hb = h2pk_ref[...]
    h2 = hb[:, :NCO]
    s2 = hb[:, NCO:NCO + 1]
    d2 = hb[:, NCO + 1:NCO + 2]
    sw = jnp.exp(_lrelu(s2 + d2))
    num = aggs + sw * h2
    den = dens + sw + 1e-16
    o = num / den + b2_ref[...]
    m = jnp.max(o, axis=1, keepdims=True)
    ls = jnp.log(jnp.sum(jnp.exp(o - m), axis=1, keepdims=True))
    out_ref[...] = o - m - ls


def kernel(x, edge_index, W1, a_src1, a_dst1, b1, W2, a_src2, a_dst2, b2):
    f32 = jnp.float32
    i32 = jnp.int32
    blk = 2000
    grid = (N // blk,)
    scmesh = plsc.VectorSubcoreMesh(core_axis_name="c", subcore_axis_name="s")

    w1r = W1.reshape(HID, HEADS, HID)
    v1s = jnp.einsum("khc,hc->kh", w1r, a_src1)
    v1d = jnp.einsum("khc,hc->kh", w1r, a_dst1)
    npd = EPAD - E
    psrc = (jnp.arange(npd, dtype=i32) * 97) % N
    pdst = N + 48 + jnp.arange(npd, dtype=i32) % 128
    srcp = jnp.concatenate([edge_index[0], psrc])
    dstp = jnp.concatenate([edge_index[1], pdst])
    z1 = jnp.zeros((R1, PK), f32)
    z2 = jnp.zeros((R1, GROW), f32)
    b1r = b1.reshape(1, HEADS * HID)
    b2r = b2.reshape(1, NCO)
    as2t = a_src2.reshape(NCO, 1)
    ad2t = a_dst2.reshape(NCO, 1)

    xpk, ad1 = pl.pallas_call(
        _tc_a_body,
        grid=grid,
        in_specs=[
            pl.BlockSpec((blk, HID), lambda i: (i, 0)),
            pl.BlockSpec((HID, HEADS), lambda i: (0, 0)),
            pl.BlockSpec((HID, HEADS), lambda i: (0, 0)),
        ],
        out_specs=[
            pl.BlockSpec((blk, XW), lambda i: (i, 0)),
            pl.BlockSpec((blk, HEADS), lambda i: (i, 0)),
        ],
        out_shape=[
            jax.ShapeDtypeStruct((N, XW), f32),
            jax.ShapeDtypeStruct((N, HEADS), f32),
        ],
    )(x, v1s, v1d)
    ad1p = jnp.concatenate([ad1, jnp.zeros((NCOV - N, HEADS), f32)])

    sc_bin = pl.kernel(
        _sc_bin_body,
        out_type=[
            jax.ShapeDtypeStruct((NWK * NBK * CAP,), i32),
            jax.ShapeDtypeStruct((NWK * NBK * CAP,), i32),
            jax.ShapeDtypeStruct((NWK * 128,), i32),
        ],
        mesh=scmesh,
        compiler_params=_SC_PARAMS,
        scratch_types=[
            pltpu.VMEM((EC,), i32),
            pltpu.VMEM((EC,), i32),
            pltpu.VMEM((CAP,), i32),
            pltpu.VMEM((CAP,), i32),
            pltpu.VMEM((128,), i32),
        ],
    )
    bsrc, bdst, counts = sc_bin(srcp, dstp)

    sc1 = pl.kernel(
        _sc1_body,
        out_type=[jax.ShapeDtypeStruct((NCOV, PK), f32)],
        mesh=scmesh,
        compiler_params=_SC_PARAMS,
        scratch_types=[
            pltpu.VMEM((R1, PK), f32),
            pltpu.VMEM((R1, HEADS), f32),
            pltpu.VMEM((NWK * 128,), i32),
            pltpu.VMEM((CH,), i32),
            pltpu.VMEM((CH,), i32),
            pltpu.VMEM((CCAP,), i32),
            pltpu.VMEM((CCAP,), i32),
            pltpu.VMEM((WG,), i32),
            pltpu.VMEM((WG,), i32),
            pltpu.VMEM((WG, XW), f32),
            pltpu.VMEM((WG * 16,), f32),
            pltpu.SemaphoreType.DMA,
        ],
    )
    (p1,) = sc1(xpk, bsrc, bdst, counts, ad1p, z1)

    h2pk, ad2row = pl.pallas_call(
        _tc_b_body,
        grid=grid,
        in_specs=[
            pl.BlockSpec((blk, PK), lambda i: (i, 0)),
            pl.BlockSpec((blk, XW), lambda i: (i, 0)),
            pl.BlockSpec((blk, HEADS), lambda i: (i, 0)),
            pl.BlockSpec((HID, HEADS * HID), lambda i: (0, 0)),
            pl.BlockSpec((1, HEADS * HID), lambda i: (0, 0)),
            pl.BlockSpec((HEADS * HID, NCO), lambda i: (0, 0)),
            pl.BlockSpec((NCO, 1), lambda i: (0, 0)),
            pl.BlockSpec((NCO, 1), lambda i: (0, 0)),
        ],
        out_specs=[
            pl.BlockSpec((blk, GROW), lambda i: (i, 0)),
            pl.BlockSpec((1, 1, blk), lambda i: (i, 0, 0)),
        ],
        out_shape=[
            jax.ShapeDtypeStruct((N, GROW), f32),
            jax.ShapeDtypeStruct((N // blk, 1, blk), f32),
        ],
    )(p1, xpk, ad1, W1, b1r, W2, as2t, ad2t)
    ad2p = jnp.concatenate([ad2row.reshape(N), jnp.zeros((NCOV - N,), f32)])

    sc2 = pl.kernel(
        _sc2_body,
        out_type=[jax.ShapeDtypeStruct((NCOV, GROW), f32)],
        mesh=scmesh,
        compiler_params=_SC_PARAMS,
        scratch_types=[
            pltpu.VMEM((R1, GROW), f32),
            pltpu.VMEM((R1,), f32),
            pltpu.VMEM((NWK * 128,), i32),
            pltpu.VMEM((CH,), i32),
            pltpu.VMEM((CH,), i32),
            pltpu.VMEM((CCAP,), i32),
            pltpu.VMEM((CCAP,), i32),
            pltpu.VMEM((WG,), i32),
            pltpu.VMEM((WG,), i32),
            pltpu.VMEM((WG, GROW), f32),
            pltpu.VMEM((WG * 16,), f32),
            pltpu.SemaphoreType.DMA,
        ],
    )
    (p2,) = sc2(h2pk, bsrc, bdst, counts, ad2p, z2)

    return pl.pallas_call(
        _tc_c_body,
        grid=grid,
        in_specs=[
            pl.BlockSpec((blk, GROW), lambda i: (i, 0)),
            pl.BlockSpec((blk, GROW), lambda i: (i, 0)),
            pl.BlockSpec((1, NCO), lambda i: (0, 0)),
        ],
        out_specs=pl.BlockSpec((blk, NCO), lambda i: (i, 0)),
        out_shape=jax.ShapeDtypeStruct((N, NCO), f32),
    )(p2, h2pk, b2r)


# speedup vs baseline: 25.4247x; 1.0804x over previous
import jax
import jax.numpy as jnp
from jax import lax
from jax.experimental import pallas as pl
from jax.experimental.pallas import tpu as pltpu
from jax.experimental.pallas import tpu_sc as plsc

N = 50000
E = 800000
HID = 64
HEADS = 4
NCO = 16

NWK = 32
EC = 25088
EPAD = NWK * EC
NBK = 7
CAP = 6144
NCOV = NBK * 8192
R1 = 256
NPASS = 7
PK = 272
GROW = 32
XW = HID + 16
WG = 128
CH = 2048
CCAP = 6272

_SC_PARAMS = pltpu.CompilerParams(
    needs_layout_passes=False, use_tc_tiling_on_sc=False)


def _lrelu(v):
    return jnp.maximum(v, 0.2 * v)


def _widx():
    return lax.axis_index("s") * 2 + lax.axis_index("c")


def _tc_a_body(x_ref, v1s_ref, v1d_ref, xpk_ref, ad1_ref):
    xb = x_ref[...]
    s = jnp.dot(xb, v1s_ref[...], preferred_element_type=jnp.float32)
    d = jnp.dot(xb, v1d_ref[...], preferred_element_type=jnp.float32)
    pad = jnp.zeros((xb.shape[0], XW - HID - HEADS), jnp.float32)
    xpk_ref[...] = jnp.concatenate([xb, s, pad], axis=1)
    ad1_ref[...] = d


def _sc_bin_body(srcp, dstp, bsrc, bdst, counts,
                 chs, chd, cs, cd, cntb):
    wid = _widx()
    lane = lax.iota(jnp.int32, 16)
    pltpu.sync_copy(srcp.at[pl.ds(wid * EC, EC)], chs)
    pltpu.sync_copy(dstp.at[pl.ds(wid * EC, EC)], chd)

    def zro(i, _):
        cntb[pl.ds(i * 16, 16)] = jnp.zeros((16,), jnp.int32)
        return 0

    lax.fori_loop(0, 8, zro, 0)

    def bkt(b, _):
        def grp(g, cur):
            s16 = chs[pl.ds(g * 16, 16)]
            d16 = chd[pl.ds(g * 16, 16)]
            m = lax.shift_right_logical(d16, 13) == b
            pc = plsc.cumsum(m.astype(jnp.int32))
            idx = cur + pc - 1
            plsc.store_scatter(cs, [idx], s16, mask=m)
            plsc.store_scatter(cd, [idx], d16, mask=m)
            return cur + jnp.max(pc)

        cur = lax.fori_loop(0, EC // 16, grp, 0)
        plsc.store_scatter(cntb, [jnp.full((16,), b * 8, jnp.int32)],
                           jnp.full((16,), cur, jnp.int32))
        pltpu.sync_copy(cs, bsrc.at[pl.ds((wid * NBK + b) * CAP, CAP)])
        pltpu.sync_copy(cd, bdst.at[pl.ds((wid * NBK + b) * CAP, CAP)])
        return 0

    lax.fori_loop(0, NBK, bkt, 0)
    pltpu.sync_copy(cntb, counts.at[pl.ds(wid * 128, 128)])


def _compact_pass(p, mylo, rng, cntb, bsrc, bdst, sbuf, dbuf, cs, cd,
                  semc, semd):
    lane = lax.iota(jnp.int32, 16)

    def src_tile(t2, cur):
        cnt = cntb[pl.ds(t2 * 128 + p * 8, 16)][0]
        lbase = (t2 * NBK + p) * CAP
        nch = (cnt + CH - 1) // CH

        def chunk(ch, cur):
            d1 = pltpu.async_copy(bsrc.at[pl.ds(lbase + ch * CH, CH)],
                                  sbuf, semc)
            d2 = pltpu.async_copy(bdst.at[pl.ds(lbase + ch * CH, CH)],
                                  dbuf, semd)
            d1.wait()
            d2.wait()

            def grp(g, cur):
                pos = ch * CH + g * 16 + lane
                s16 = sbuf[pl.ds(g * 16, 16)]
                d16 = dbuf[pl.ds(g * 16, 16)]
                dl = d16 - mylo
                m = (pos < cnt) & (d16 >= mylo) & (d16 < mylo + rng)
                pc = plsc.cumsum(m.astype(jnp.int32))
                idx = cur + pc - 1
                plsc.store_scatter(cs, [idx], s16, mask=m)
                plsc.store_scatter(cd, [idx], dl, mask=m)
                return cur + jnp.max(pc)

            return lax.fori_loop(0, CH // 16, grp, cur)

        return lax.fori_loop(0, nch, chunk, cur)

    return lax.fori_loop(0, NWK, src_tile, 0)


def _sc1_body(xpk, bsrc, bdst, counts, ad1p, z1, p1,
              acc, ad1t, cntb, sbuf, dbuf, cs, cd,
              cswa, cdwa, cswb, cdwb, gbufa, gbufb, wbuf,
              sema, semb, semc, semd):
    wid = _widx()
    lane = lax.iota(jnp.int32, 16)
    pltpu.sync_copy(counts, cntb)

    def ini(i, _):
        z = jnp.zeros((16,), jnp.int32)
        cs[pl.ds(i * 16, 16)] = z
        cd[pl.ds(i * 16, 16)] = z
        wbuf[pl.ds(i * 16, 16)] = jnp.zeros((16,), jnp.float32)
        return 0

    lax.fori_loop(0, CCAP // 16, ini, 0)

    def do_pass(p, _):
        mylo = p * 8192 + wid * R1
        pltpu.sync_copy(z1, acc)
        pltpu.sync_copy(ad1p.at[pl.ds(mylo, R1)], ad1t)
        k_cnt = _compact_pass(p, mylo, R1, cntb, bsrc, bdst,
                              sbuf, dbuf, cs, cd, semc, semd)
        n_sub = (k_cnt + WG - 1) // WG

        def issue(j, cswx, cdwx, gbufx, semx):
            off = jnp.minimum(j * WG, CCAP - WG)
            for q in range(WG // 16):
                cswx[pl.ds(q * 16, 16)] = cs[pl.ds(off + q * 16, 16)]
                cdwx[pl.ds(q * 16, 16)] = cd[pl.ds(off + q * 16, 16)]
            pltpu.async_copy(xpk.at[cswx], gbufx, semx)

        def process(j, cdwx, gbufx):
            def wgrp(q, _):
                row = q * 16 + lane
                dl = cdwx[pl.ds(q * 16, 16)]
                live = (j * WG + row) < k_cnt
                for h in range(HEADS):
                    a_s = plsc.load_gather(
                        gbufx, [row, jnp.full((16,), HID + h, jnp.int32)])
                    a_d = plsc.load_gather(
                        ad1t, [dl, jnp.full((16,), h, jnp.int32)])
                    wv = jnp.exp(_lrelu(a_s + a_d))
                    wv = jnp.where(live, wv, 0.0)
                    plsc.store_scatter(wbuf, [row * 16 + h], wv)
                return 0

            lax.fori_loop(0, WG // 16, wgrp, 0)

            def egrp(g, _):
                dl16 = cdwx[pl.ds(g * 16, 16)]
                for kk in range(16):
                    dl = dl16[kk]
                    row = g * 16 + kk
                    wrow = wbuf[pl.ds(row * 16, 16)]
                    wsps = [jnp.full((16,), wrow[h]) for h in range(HEADS)]
                    for jx in range(HID // 16):
                        xv = gbufx[row, pl.ds(jx * 16, 16)]
                        for h in range(HEADS):
                            col = h * HID + jx * 16
                            plsc.addupdate(acc.at[dl, pl.ds(col, 16)],
                                           wsps[h] * xv)
                    plsc.addupdate(acc.at[dl, pl.ds(4 * HID, 16)], wrow)
                return 0

            lax.fori_loop(0, WG // 16, egrp, 0)

        issue(0, cswa, cdwa, gbufa, sema)

        def two(t, _):
            j0 = 2 * t
            issue(j0 + 1, cswb, cdwb, gbufb, semb)
            pltpu.make_async_copy(xpk.at[cswa], gbufa, sema).wait()
            process(j0, cdwa, gbufa)
            issue(j0 + 2, cswa, cdwa, gbufa, sema)
            pltpu.make_async_copy(xpk.at[cswb], gbufb, semb).wait()
            process(j0 + 1, cdwb, gbufb)
            return 0

        lax.fori_loop(0, (n_sub + 1) // 2, two, 0)
        pltpu.make_async_copy(xpk.at[cswa], gbufa, sema).wait()
        pltpu.sync_copy(acc, p1.at[pl.ds(mylo, R1)])
        return 0

    lax.fori_loop(0, NPASS, do_pass, 0)


def _tc_b_body(p1_ref, xpk_ref, ad1_ref, w1_ref, b1_ref, w2_ref,
               as2t_ref, ad2t_ref, h2pk_ref, ad2row_ref):
    ps = p1_ref[...]
    xb = xpk_ref[:, :HID]
    as1 = xpk_ref[:, HID:HID + HEADS]
    ad1 = ad1_ref[...]
    sw = jnp.exp(_lrelu(as1 + ad1))
    outs = []
    for h in range(HEADS):
        ah = ps[:, h * HID:(h + 1) * HID] + sw[:, h:h + 1] * xb
        dh = ps[:, 4 * HID + h:4 * HID + h + 1] + sw[:, h:h + 1] + 1e-16
        outs.append(jnp.dot(ah / dh, w1_ref[:, h * HID:(h + 1) * HID],
                            preferred_element_type=jnp.float32))
    o1 = jnp.concatenate(outs, axis=1) + b1_ref[...]
    he = jnp.where(o1 > 0, o1, jnp.exp(jnp.minimum(o1, 0.0)) - 1.0)
    h2 = jnp.dot(he, w2_ref[...], preferred_element_type=jnp.float32)
    s2 = jnp.dot(h2, as2t_ref[...], preferred_element_type=jnp.float32)
    d2 = jnp.dot(h2, ad2t_ref[...], preferred_element_type=jnp.float32)
    pad = jnp.zeros((h2.shape[0], GROW - NCO - 2), jnp.float32)
    h2pk_ref[...] = jnp.concatenate([h2, s2, d2, pad], axis=1)
    ad2row_ref[...] = jnp.reshape(
        lax.dot_general(jnp.transpose(ad2t_ref[...]), h2,
                        (((1,), (1,)), ((), ())),
                        preferred_element_type=jnp.float32),
        (1, 1, h2.shape[0]))


def _sc2_body(h2pk, bsrc, bdst, counts, ad2p, z2, p2,
              acc, ad2t, cntb, sbuf, dbuf, cs, cd,
              cswa, cdwa, cswb, cdwb, gbufa, gbufb, wbuf,
              sema, semb, semc, semd):
    wid = _widx()
    lane = lax.iota(jnp.int32, 16)
    pltpu.sync_copy(counts, cntb)

    def ini(i, _):
        z = jnp.zeros((16,), jnp.int32)
        cs[pl.ds(i * 16, 16)] = z
        cd[pl.ds(i * 16, 16)] = z
        wbuf[pl.ds(i * 16, 16)] = jnp.zeros((16,), jnp.float32)
        return 0

    lax.fori_loop(0, CCAP // 16, ini, 0)

    def do_pass(p, _):
        mylo = p * 8192 + wid * R1
        pltpu.sync_copy(z2, acc)
        pltpu.sync_copy(ad2p.at[pl.ds(mylo, R1)], ad2t)
        k_cnt = _compact_pass(p, mylo, R1, cntb, bsrc, bdst,
                              sbuf, dbuf, cs, cd, semc, semd)
        n_sub = (k_cnt + WG - 1) // WG

        def issue(j, cswx, cdwx, gbufx, semx):
            off = jnp.minimum(j * WG, CCAP - WG)
            for q in range(WG // 16):
                cswx[pl.ds(q * 16, 16)] = cs[pl.ds(off + q * 16, 16)]
                cdwx[pl.ds(q * 16, 16)] = cd[pl.ds(off + q * 16, 16)]
            pltpu.async_copy(h2pk.at[cswx], gbufx, semx)

        def process(j, cdwx, gbufx):
            def wgrp(q, _):
                row = q * 16 + lane
                dl = cdwx[pl.ds(q * 16, 16)]
                live = (j * WG + row) < k_cnt
                a_s = plsc.load_gather(
                    gbufx, [row, jnp.full((16,), NCO, jnp.int32)])
                a_d = plsc.load_gather(ad2t, [dl])
                wv = jnp.exp(_lrelu(a_s + a_d))
                wv = jnp.where(live, wv, 0.0)
                plsc.store_scatter(wbuf, [row * 16], wv)
                return 0

            lax.fori_loop(0, WG // 16, wgrp, 0)

            def egrp(g, _):
                dl16 = cdwx[pl.ds(g * 16, 16)]
                for kk in range(16):
                    dl = dl16[kk]
                    row = g * 16 + kk
                    wrow = wbuf[pl.ds(row * 16, 16)]
                    wsp = jnp.full((16,), wrow[0])
                    xv = gbufx[row, pl.ds(0, 16)]
                    plsc.addupdate(acc.at[dl, pl.ds(0, 16)], wsp * xv)
                    plsc.addupdate(acc.at[dl, pl.ds(16, 16)], wrow)
                return 0

            lax.fori_loop(0, WG // 16, egrp, 0)

        issue(0, cswa, cdwa, gbufa, sema)

        def two(t, _):
            j0 = 2 * t
            issue(j0 + 1, cswb, cdwb, gbufb, semb)
            pltpu.make_async_copy(h2pk.at[cswa], gbufa, sema).wait()
            process(j0, cdwa, gbufa)
            issue(j0 + 2, cswa, cdwa, gbufa, sema)
            pltpu.make_async_copy(h2pk.at[cswb], gbufb, semb).wait()
            process(j0 + 1, cdwb, gbufb)
            return 0

        lax.fori_loop(0, (n_sub + 1) // 2, two, 0)
        pltpu.make_async_copy(h2pk.at[cswa], gbufa, sema).wait()
        pltpu.sync_copy(acc, p2.at[pl.ds(mylo, R1)])
        return 0

    lax.fori_loop(0, NPASS, do_pass, 0)


def _tc_c_body(p2_ref, h2pk_ref, b2_ref, out_ref):
    pb = p2_ref[...]
    aggs = pb[:, :NCO]
    dens = pb[:, NCO:NCO + 1]
    hb = h2pk_ref[...]
    h2 = hb[:, :NCO]
    s2 = hb[:, NCO:NCO + 1]
    d2 = hb[:, NCO + 1:NCO + 2]
    sw = jnp.exp(_lrelu(s2 + d2))
    num = aggs + sw * h2
    den = dens + sw + 1e-16
    o = num / den + b2_ref[...]
    m = jnp.max(o, axis=1, keepdims=True)
    ls = jnp.log(jnp.sum(jnp.exp(o - m), axis=1, keepdims=True))
    out_ref[...] = o - m - ls


def kernel(x, edge_index, W1, a_src1, a_dst1, b1, W2, a_src2, a_dst2, b2):
    f32 = jnp.float32
    i32 = jnp.int32
    blk = 2000
    grid = (N // blk,)
    scmesh = plsc.VectorSubcoreMesh(core_axis_name="c", subcore_axis_name="s")

    w1r = W1.reshape(HID, HEADS, HID)
    v1s = jnp.einsum("khc,hc->kh", w1r, a_src1)
    v1d = jnp.einsum("khc,hc->kh", w1r, a_dst1)
    npd = EPAD - E
    psrc = (jnp.arange(npd, dtype=i32) * 97) % N
    pdst = N + 48 + jnp.arange(npd, dtype=i32) % 128
    srcp = jnp.concatenate([edge_index[0], psrc])
    dstp = jnp.concatenate([edge_index[1], pdst])
    z1 = jnp.zeros((R1, PK), f32)
    z2 = jnp.zeros((R1, GROW), f32)
    b1r = b1.reshape(1, HEADS * HID)
    b2r = b2.reshape(1, NCO)
    as2t = a_src2.reshape(NCO, 1)
    ad2t = a_dst2.reshape(NCO, 1)

    xpk, ad1 = pl.pallas_call(
        _tc_a_body,
        grid=grid,
        in_specs=[
            pl.BlockSpec((blk, HID), lambda i: (i, 0)),
            pl.BlockSpec((HID, HEADS), lambda i: (0, 0)),
            pl.BlockSpec((HID, HEADS), lambda i: (0, 0)),
        ],
        out_specs=[
            pl.BlockSpec((blk, XW), lambda i: (i, 0)),
            pl.BlockSpec((blk, HEADS), lambda i: (i, 0)),
        ],
        out_shape=[
            jax.ShapeDtypeStruct((N, XW), f32),
            jax.ShapeDtypeStruct((N, HEADS), f32),
        ],
    )(x, v1s, v1d)
    ad1p = jnp.concatenate([ad1, jnp.zeros((NCOV - N, HEADS), f32)])

    sc_bin = pl.kernel(
        _sc_bin_body,
        out_type=[
            jax.ShapeDtypeStruct((NWK * NBK * CAP,), i32),
            jax.ShapeDtypeStruct((NWK * NBK * CAP,), i32),
            jax.ShapeDtypeStruct((NWK * 128,), i32),
        ],
        mesh=scmesh,
        compiler_params=_SC_PARAMS,
        scratch_types=[
            pltpu.VMEM((EC,), i32),
            pltpu.VMEM((EC,), i32),
            pltpu.VMEM((CAP,), i32),
            pltpu.VMEM((CAP,), i32),
            pltpu.VMEM((128,), i32),
        ],
    )
    bsrc, bdst, counts = sc_bin(srcp, dstp)

    sc1 = pl.kernel(
        _sc1_body,
        out_type=[jax.ShapeDtypeStruct((NCOV, PK), f32)],
        mesh=scmesh,
        compiler_params=_SC_PARAMS,
        scratch_types=[
            pltpu.VMEM((R1, PK), f32),
            pltpu.VMEM((R1, HEADS), f32),
            pltpu.VMEM((NWK * 128,), i32),
            pltpu.VMEM((CH,), i32),
            pltpu.VMEM((CH,), i32),
            pltpu.VMEM((CCAP,), i32),
            pltpu.VMEM((CCAP,), i32),
            pltpu.VMEM((WG,), i32),
            pltpu.VMEM((WG,), i32),
            pltpu.VMEM((WG,), i32),
            pltpu.VMEM((WG,), i32),
            pltpu.VMEM((WG, XW), f32),
            pltpu.VMEM((WG, XW), f32),
            pltpu.VMEM((WG * 16,), f32),
            pltpu.SemaphoreType.DMA,
            pltpu.SemaphoreType.DMA,
            pltpu.SemaphoreType.DMA,
            pltpu.SemaphoreType.DMA,
        ],
    )
    (p1,) = sc1(xpk, bsrc, bdst, counts, ad1p, z1)

    h2pk, ad2row = pl.pallas_call(
        _tc_b_body,
        grid=grid,
        in_specs=[
            pl.BlockSpec((blk, PK), lambda i: (i, 0)),
            pl.BlockSpec((blk, XW), lambda i: (i, 0)),
            pl.BlockSpec((blk, HEADS), lambda i: (i, 0)),
            pl.BlockSpec((HID, HEADS * HID), lambda i: (0, 0)),
            pl.BlockSpec((1, HEADS * HID), lambda i: (0, 0)),
            pl.BlockSpec((HEADS * HID, NCO), lambda i: (0, 0)),
            pl.BlockSpec((NCO, 1), lambda i: (0, 0)),
            pl.BlockSpec((NCO, 1), lambda i: (0, 0)),
        ],
        out_specs=[
            pl.BlockSpec((blk, GROW), lambda i: (i, 0)),
            pl.BlockSpec((1, 1, blk), lambda i: (i, 0, 0)),
        ],
        out_shape=[
            jax.ShapeDtypeStruct((N, GROW), f32),
            jax.ShapeDtypeStruct((N // blk, 1, blk), f32),
        ],
    )(p1, xpk, ad1, W1, b1r, W2, as2t, ad2t)
    ad2p = jnp.concatenate([ad2row.reshape(N), jnp.zeros((NCOV - N,), f32)])

    sc2 = pl.kernel(
        _sc2_body,
        out_type=[jax.ShapeDtypeStruct((NCOV, GROW), f32)],
        mesh=scmesh,
        compiler_params=_SC_PARAMS,
        scratch_types=[
            pltpu.VMEM((R1, GROW), f32),
            pltpu.VMEM((R1,), f32),
            pltpu.VMEM((NWK * 128,), i32),
            pltpu.VMEM((CH,), i32),
            pltpu.VMEM((CH,), i32),
            pltpu.VMEM((CCAP,), i32),
            pltpu.VMEM((CCAP,), i32),
            pltpu.VMEM((WG,), i32),
            pltpu.VMEM((WG,), i32),
            pltpu.VMEM((WG,), i32),
            pltpu.VMEM((WG,), i32),
            pltpu.VMEM((WG, GROW), f32),
            pltpu.VMEM((WG, GROW), f32),
            pltpu.VMEM((WG * 16,), f32),
            pltpu.SemaphoreType.DMA,
            pltpu.SemaphoreType.DMA,
            pltpu.SemaphoreType.DMA,
            pltpu.SemaphoreType.DMA,
        ],
    )
    (p2,) = sc2(h2pk, bsrc, bdst, counts, ad2p, z2)

    return pl.pallas_call(
        _tc_c_body,
        grid=grid,
        in_specs=[
            pl.BlockSpec((blk, GROW), lambda i: (i, 0)),
            pl.BlockSpec((blk, GROW), lambda i: (i, 0)),
            pl.BlockSpec((1, NCO), lambda i: (0, 0)),
        ],
        out_specs=pl.BlockSpec((blk, NCO), lambda i: (i, 0)),
        out_shape=jax.ShapeDtypeStruct((N, NCO), f32),
    )(p2, h2pk, b2r)


# speedup vs baseline: 25.4452x; 1.0008x over previous
import jax
import jax.numpy as jnp
from jax import lax
from jax.experimental import pallas as pl
from jax.experimental.pallas import tpu as pltpu
from jax.experimental.pallas import tpu_sc as plsc

N = 50000
E = 800000
HID = 64
HEADS = 4
NCO = 16

NWK = 32
EC = 25088
EPAD = NWK * EC
NBK = 7
CAP = 6144
NCOV = NBK * 8192
R1 = 256
NPASS = 7
PK = 272
GROW = 32
XW = HID + 16
WG = 128
CH = 2048
CCAP = 6272

_SC_PARAMS = pltpu.CompilerParams(
    needs_layout_passes=False, use_tc_tiling_on_sc=False)


def _lrelu(v):
    return jnp.maximum(v, 0.2 * v)


def _widx():
    return lax.axis_index("s") * 2 + lax.axis_index("c")


def _tc_a_body(x_ref, v1s_ref, v1d_ref, xpk_ref, ad1_ref):
    xb = x_ref[...]
    s = jnp.dot(xb, v1s_ref[...], preferred_element_type=jnp.float32)
    d = jnp.dot(xb, v1d_ref[...], preferred_element_type=jnp.float32)
    pad = jnp.zeros((xb.shape[0], XW - HID - HEADS), jnp.float32)
    xpk_ref[...] = jnp.concatenate([xb, s, pad], axis=1)
    ad1_ref[...] = d


def _sc_bin_body(srcp, dstp, bsrc, bdst, counts,
                 chs, chd, cs, cd, cntb):
    wid = _widx()
    lane = lax.iota(jnp.int32, 16)
    pltpu.sync_copy(srcp.at[pl.ds(wid * EC, EC)], chs)
    pltpu.sync_copy(dstp.at[pl.ds(wid * EC, EC)], chd)

    def zro(i, _):
        cntb[pl.ds(i * 16, 16)] = jnp.zeros((16,), jnp.int32)
        return 0

    lax.fori_loop(0, 8, zro, 0)

    def bkt(b, _):
        def grp(g, cur):
            s16 = chs[pl.ds(g * 16, 16)]
            d16 = chd[pl.ds(g * 16, 16)]
            m = lax.shift_right_logical(d16, 13) == b
            pc = plsc.cumsum(m.astype(jnp.int32))
            idx = cur + pc - 1
            plsc.store_scatter(cs, [idx], s16, mask=m)
            plsc.store_scatter(cd, [idx], d16, mask=m)
            return cur + jnp.max(pc)

        cur = lax.fori_loop(0, EC // 16, grp, 0)
        plsc.store_scatter(cntb, [jnp.full((16,), b * 8, jnp.int32)],
                           jnp.full((16,), cur, jnp.int32))
        pltpu.sync_copy(cs, bsrc.at[pl.ds((wid * NBK + b) * CAP, CAP)])
        pltpu.sync_copy(cd, bdst.at[pl.ds((wid * NBK + b) * CAP, CAP)])
        return 0

    lax.fori_loop(0, NBK, bkt, 0)
    pltpu.sync_copy(cntb, counts.at[pl.ds(wid * 128, 128)])


def _compact_pass(bk, mylo, rng, cntb, bsrc, bdst, sbuf, dbuf, cs, cd,
                  semc, semd):
    lane = lax.iota(jnp.int32, 16)

    def src_tile(t2, cur):
        cnt = cntb[pl.ds(t2 * 128 + bk * 8, 16)][0]
        lbase = (t2 * NBK + bk) * CAP
        nch = (cnt + CH - 1) // CH

        def chunk(ch, cur):
            d1 = pltpu.async_copy(bsrc.at[pl.ds(lbase + ch * CH, CH)],
                                  sbuf, semc)
            d2 = pltpu.async_copy(bdst.at[pl.ds(lbase + ch * CH, CH)],
                                  dbuf, semd)
            d1.wait()
            d2.wait()

            def grp(g, cur):
                pos = ch * CH + g * 16 + lane
                s16 = sbuf[pl.ds(g * 16, 16)]
                d16 = dbuf[pl.ds(g * 16, 16)]
                dl = d16 - mylo
                m = (pos < cnt) & (d16 >= mylo) & (d16 < mylo + rng)
                pc = plsc.cumsum(m.astype(jnp.int32))
                idx = cur + pc - 1
                plsc.store_scatter(cs, [idx], s16, mask=m)
                plsc.store_scatter(cd, [idx], dl, mask=m)
                return cur + jnp.max(pc)

            return lax.fori_loop(0, CH // 16, grp, cur)

        return lax.fori_loop(0, nch, chunk, cur)

    return lax.fori_loop(0, NWK, src_tile, 0)


def _sc1_body(xpk, bsrc, bdst, counts, ad1p, z1, p1,
              acc, ad1t, cntb, sbuf, dbuf, cs, cd,
              cswa, cdwa, cswb, cdwb, gbufa, gbufb, wbuf,
              sema, semb, semc, semd):
    wid = _widx()
    lane = lax.iota(jnp.int32, 16)
    pltpu.sync_copy(counts, cntb)

    def ini(i, _):
        z = jnp.zeros((16,), jnp.int32)
        cs[pl.ds(i * 16, 16)] = z
        cd[pl.ds(i * 16, 16)] = z
        wbuf[pl.ds(i * 16, 16)] = jnp.zeros((16,), jnp.float32)
        return 0

    lax.fori_loop(0, CCAP // 16, ini, 0)

    def do_pass(p, _):
        mylo = p * 8192 + wid * R1
        bk = p
        pltpu.sync_copy(z1, acc)
        pltpu.sync_copy(ad1p.at[pl.ds(mylo, R1)], ad1t)
        k_cnt = _compact_pass(bk, mylo, R1, cntb, bsrc, bdst,
                              sbuf, dbuf, cs, cd, semc, semd)
        n_sub = (k_cnt + WG - 1) // WG

        def issue(j, cswx, cdwx, gbufx, semx):
            off = jnp.minimum(j * WG, CCAP - WG)
            for q in range(WG // 16):
                cswx[pl.ds(q * 16, 16)] = cs[pl.ds(off + q * 16, 16)]
                cdwx[pl.ds(q * 16, 16)] = cd[pl.ds(off + q * 16, 16)]
            pltpu.async_copy(xpk.at[cswx], gbufx, semx)

        def process(j, cdwx, gbufx):
            def wgrp(q, _):
                row = q * 16 + lane
                dl = cdwx[pl.ds(q * 16, 16)]
                live = (j * WG + row) < k_cnt
                for h in range(HEADS):
                    a_s = plsc.load_gather(
                        gbufx, [row, jnp.full((16,), HID + h, jnp.int32)])
                    a_d = plsc.load_gather(
                        ad1t, [dl, jnp.full((16,), h, jnp.int32)])
                    wv = jnp.exp(_lrelu(a_s + a_d))
                    wv = jnp.where(live, wv, 0.0)
                    plsc.store_scatter(wbuf, [row * 16 + h], wv)
                return 0

            lax.fori_loop(0, WG // 16, wgrp, 0)

            def egrp(g, _):
                dl16 = cdwx[pl.ds(g * 16, 16)]
                for kk in range(16):
                    dl = dl16[kk]
                    row = g * 16 + kk
                    wrow = wbuf[pl.ds(row * 16, 16)]
                    wsps = [jnp.full((16,), wrow[h]) for h in range(HEADS)]
                    for jx in range(HID // 16):
                        xv = gbufx[row, pl.ds(jx * 16, 16)]
                        for h in range(HEADS):
                            col = h * HID + jx * 16
                            plsc.addupdate(acc.at[dl, pl.ds(col, 16)],
                                           wsps[h] * xv)
                    plsc.addupdate(acc.at[dl, pl.ds(4 * HID, 16)], wrow)
                return 0

            lax.fori_loop(0, WG // 16, egrp, 0)

        issue(0, cswa, cdwa, gbufa, sema)

        def two(t, _):
            j0 = 2 * t
            issue(j0 + 1, cswb, cdwb, gbufb, semb)
            pltpu.make_async_copy(xpk.at[cswa], gbufa, sema).wait()
            process(j0, cdwa, gbufa)
            issue(j0 + 2, cswa, cdwa, gbufa, sema)
            pltpu.make_async_copy(xpk.at[cswb], gbufb, semb).wait()
            process(j0 + 1, cdwb, gbufb)
            return 0

        lax.fori_loop(0, (n_sub + 1) // 2, two, 0)
        pltpu.make_async_copy(xpk.at[cswa], gbufa, sema).wait()
        pltpu.sync_copy(acc, p1.at[pl.ds(mylo, R1)])
        return 0

    lax.fori_loop(0, NPASS, do_pass, 0)


def _tc_b_body(p1_ref, xpk_ref, ad1_ref, w1_ref, b1_ref, w2_ref,
               as2t_ref, ad2t_ref, h2pk_ref, ad2row_ref):
    ps = p1_ref[...]
    xb = xpk_ref[:, :HID]
    as1 = xpk_ref[:, HID:HID + HEADS]
    ad1 = ad1_ref[...]
    sw = jnp.exp(_lrelu(as1 + ad1))
    outs = []
    for h in range(HEADS):
        ah = ps[:, h * HID:(h + 1) * HID] + sw[:, h:h + 1] * xb
        dh = ps[:, 4 * HID + h:4 * HID + h + 1] + sw[:, h:h + 1] + 1e-16
        outs.append(jnp.dot(ah / dh, w1_ref[:, h * HID:(h + 1) * HID],
                            preferred_element_type=jnp.float32))
    o1 = jnp.concatenate(outs, axis=1) + b1_ref[...]
    he = jnp.where(o1 > 0, o1, jnp.exp(jnp.minimum(o1, 0.0)) - 1.0)
    h2 = jnp.dot(he, w2_ref[...], preferred_element_type=jnp.float32)
    s2 = jnp.dot(h2, as2t_ref[...], preferred_element_type=jnp.float32)
    d2 = jnp.dot(h2, ad2t_ref[...], preferred_element_type=jnp.float32)
    pad = jnp.zeros((h2.shape[0], GROW - NCO - 2), jnp.float32)
    h2pk_ref[...] = jnp.concatenate([h2, s2, d2, pad], axis=1)
    ad2row_ref[...] = jnp.reshape(
        lax.dot_general(jnp.transpose(ad2t_ref[...]), h2,
                        (((1,), (1,)), ((), ())),
                        preferred_element_type=jnp.float32),
        (1, 1, h2.shape[0]))


def _sc2_body(h2pk, bsrc, bdst, counts, ad2p, z2, p2,
              acc, ad2t, cntb, sbuf, dbuf, cs, cd,
              cswa, cdwa, cswb, cdwb, gbufa, gbufb, wbuf,
              sema, semb, semc, semd):
    wid = _widx()
    lane = lax.iota(jnp.int32, 16)
    pltpu.sync_copy(counts, cntb)

    def ini(i, _):
        z = jnp.zeros((16,), jnp.int32)
        cs[pl.ds(i * 16, 16)] = z
        cd[pl.ds(i * 16, 16)] = z
        wbuf[pl.ds(i * 16, 16)] = jnp.zeros((16,), jnp.float32)
        return 0

    lax.fori_loop(0, CCAP // 16, ini, 0)

    def do_pass(p, _):
        mylo = p * 8192 + wid * R1
        bk = p
        pltpu.sync_copy(z2, acc)
        pltpu.sync_copy(ad2p.at[pl.ds(mylo, R1)], ad2t)
        k_cnt = _compact_pass(bk, mylo, R1, cntb, bsrc, bdst,
                              sbuf, dbuf, cs, cd, semc, semd)
        n_sub = (k_cnt + WG - 1) // WG

        def issue(j, cswx, cdwx, gbufx, semx):
            off = jnp.minimum(j * WG, CCAP - WG)
            for q in range(WG // 16):
                cswx[pl.ds(q * 16, 16)] = cs[pl.ds(off + q * 16, 16)]
                cdwx[pl.ds(q * 16, 16)] = cd[pl.ds(off + q * 16, 16)]
            pltpu.async_copy(h2pk.at[cswx], gbufx, semx)

        def process(j, cdwx, gbufx):
            def wgrp(q, _):
                row = q * 16 + lane
                dl = cdwx[pl.ds(q * 16, 16)]
                live = (j * WG + row) < k_cnt
                a_s = plsc.load_gather(
                    gbufx, [row, jnp.full((16,), NCO, jnp.int32)])
                a_d = plsc.load_gather(ad2t, [dl])
                wv = jnp.exp(_lrelu(a_s + a_d))
                wv = jnp.where(live, wv, 0.0)
                plsc.store_scatter(wbuf, [row * 16], wv)
                return 0

            lax.fori_loop(0, WG // 16, wgrp, 0)

            def egrp(g, _):
                dl16 = cdwx[pl.ds(g * 16, 16)]
                for kk in range(16):
                    dl = dl16[kk]
                    row = g * 16 + kk
                    wrow = wbuf[pl.ds(row * 16, 16)]
                    wsp = jnp.full((16,), wrow[0])
                    xv = gbufx[row, pl.ds(0, 16)]
                    plsc.addupdate(acc.at[dl, pl.ds(0, 16)], wsp * xv)
                    plsc.addupdate(acc.at[dl, pl.ds(16, 16)], wrow)
                return 0

            lax.fori_loop(0, WG // 16, egrp, 0)

        issue(0, cswa, cdwa, gbufa, sema)

        def two(t, _):
            j0 = 2 * t
            issue(j0 + 1, cswb, cdwb, gbufb, semb)
            pltpu.make_async_copy(h2pk.at[cswa], gbufa, sema).wait()
            process(j0, cdwa, gbufa)
            issue(j0 + 2, cswa, cdwa, gbufa, sema)
            pltpu.make_async_copy(h2pk.at[cswb], gbufb, semb).wait()
            process(j0 + 1, cdwb, gbufb)
            return 0

        lax.fori_loop(0, (n_sub + 1) // 2, two, 0)
        pltpu.make_async_copy(h2pk.at[cswa], gbufa, sema).wait()
        pltpu.sync_copy(acc, p2.at[pl.ds(mylo, R1)])
        return 0

    lax.fori_loop(0, NPASS, do_pass, 0)


def _tc_c_body(p2_ref, h2pk_ref, b2_ref, out_ref):
    pb = p2_ref[...]
    aggs = pb[:, :NCO]
    dens = pb[:, NCO:NCO + 1]
    hb = h2pk_ref[...]
    h2 = hb[:, :NCO]
    s2 = hb[:, NCO:NCO + 1]
    d2 = hb[:, NCO + 1:NCO + 2]
    sw = jnp.exp(_lrelu(s2 + d2))
    num = aggs + sw * h2
    den = dens + sw + 1e-16
    o = num / den + b2_ref[...]
    m = jnp.max(o, axis=1, keepdims=True)
    ls = jnp.log(jnp.sum(jnp.exp(o - m), axis=1, keepdims=True))
    out_ref[...] = o - m - ls


def kernel(x, edge_index, W1, a_src1, a_dst1, b1, W2, a_src2, a_dst2, b2):
    f32 = jnp.float32
    i32 = jnp.int32
    blk = 2000
    grid = (N // blk,)
    scmesh = plsc.VectorSubcoreMesh(core_axis_name="c", subcore_axis_name="s")

    w1r = W1.reshape(HID, HEADS, HID)
    v1s = jnp.einsum("khc,hc->kh", w1r, a_src1)
    v1d = jnp.einsum("khc,hc->kh", w1r, a_dst1)
    npd = EPAD - E
    psrc = (jnp.arange(npd, dtype=i32) * 97) % N
    pdst = N + 48 + jnp.arange(npd, dtype=i32) % 128
    srcp = jnp.concatenate([edge_index[0], psrc])
    dstp = jnp.concatenate([edge_index[1], pdst])
    z1 = jnp.zeros((R1, PK), f32)
    z2 = jnp.zeros((R1, GROW), f32)
    b1r = b1.reshape(1, HEADS * HID)
    b2r = b2.reshape(1, NCO)
    as2t = a_src2.reshape(NCO, 1)
    ad2t = a_dst2.reshape(NCO, 1)

    xpk, ad1 = pl.pallas_call(
        _tc_a_body,
        grid=grid,
        in_specs=[
            pl.BlockSpec((blk, HID), lambda i: (i, 0)),
            pl.BlockSpec((HID, HEADS), lambda i: (0, 0)),
            pl.BlockSpec((HID, HEADS), lambda i: (0, 0)),
        ],
        out_specs=[
            pl.BlockSpec((blk, XW), lambda i: (i, 0)),
            pl.BlockSpec((blk, HEADS), lambda i: (i, 0)),
        ],
        out_shape=[
            jax.ShapeDtypeStruct((N, XW), f32),
            jax.ShapeDtypeStruct((N, HEADS), f32),
        ],
    )(x, v1s, v1d)
    ad1p = jnp.concatenate([ad1, jnp.zeros((NCOV - N, HEADS), f32)])

    sc_bin = pl.kernel(
        _sc_bin_body,
        out_type=[
            jax.ShapeDtypeStruct((NWK * NBK * CAP,), i32),
            jax.ShapeDtypeStruct((NWK * NBK * CAP,), i32),
            jax.ShapeDtypeStruct((NWK * 128,), i32),
        ],
        mesh=scmesh,
        compiler_params=_SC_PARAMS,
        scratch_types=[
            pltpu.VMEM((EC,), i32),
            pltpu.VMEM((EC,), i32),
            pltpu.VMEM((CAP,), i32),
            pltpu.VMEM((CAP,), i32),
            pltpu.VMEM((128,), i32),
        ],
    )
    bsrc, bdst, counts = sc_bin(srcp, dstp)

    sc1 = pl.kernel(
        _sc1_body,
        out_type=[jax.ShapeDtypeStruct((NCOV, PK), f32)],
        mesh=scmesh,
        compiler_params=_SC_PARAMS,
        scratch_types=[
            pltpu.VMEM((R1, PK), f32),
            pltpu.VMEM((R1, HEADS), f32),
            pltpu.VMEM((NWK * 128,), i32),
            pltpu.VMEM((CH,), i32),
            pltpu.VMEM((CH,), i32),
            pltpu.VMEM((CCAP,), i32),
            pltpu.VMEM((CCAP,), i32),
            pltpu.VMEM((WG,), i32),
            pltpu.VMEM((WG,), i32),
            pltpu.VMEM((WG,), i32),
            pltpu.VMEM((WG,), i32),
            pltpu.VMEM((WG, XW), f32),
            pltpu.VMEM((WG, XW), f32),
            pltpu.VMEM((WG * 16,), f32),
            pltpu.SemaphoreType.DMA,
            pltpu.SemaphoreType.DMA,
            pltpu.SemaphoreType.DMA,
            pltpu.SemaphoreType.DMA,
        ],
    )
    (p1,) = sc1(xpk, bsrc, bdst, counts, ad1p, z1)

    h2pk, ad2row = pl.pallas_call(
        _tc_b_body,
        grid=grid,
        in_specs=[
            pl.BlockSpec((blk, PK), lambda i: (i, 0)),
            pl.BlockSpec((blk, XW), lambda i: (i, 0)),
            pl.BlockSpec((blk, HEADS), lambda i: (i, 0)),
            pl.BlockSpec((HID, HEADS * HID), lambda i: (0, 0)),
            pl.BlockSpec((1, HEADS * HID), lambda i: (0, 0)),
            pl.BlockSpec((HEADS * HID, NCO), lambda i: (0, 0)),
            pl.BlockSpec((NCO, 1), lambda i: (0, 0)),
            pl.BlockSpec((NCO, 1), lambda i: (0, 0)),
        ],
        out_specs=[
            pl.BlockSpec((blk, GROW), lambda i: (i, 0)),
            pl.BlockSpec((1, 1, blk), lambda i: (i, 0, 0)),
        ],
        out_shape=[
            jax.ShapeDtypeStruct((N, GROW), f32),
            jax.ShapeDtypeStruct((N // blk, 1, blk), f32),
        ],
    )(p1, xpk, ad1, W1, b1r, W2, as2t, ad2t)
    ad2p = jnp.concatenate([ad2row.reshape(N), jnp.zeros((NCOV - N,), f32)])

    sc2 = pl.kernel(
        _sc2_body,
        out_type=[jax.ShapeDtypeStruct((NCOV, GROW), f32)],
        mesh=scmesh,
        compiler_params=_SC_PARAMS,
        scratch_types=[
            pltpu.VMEM((R1, GROW), f32),
            pltpu.VMEM((R1,), f32),
            pltpu.VMEM((NWK * 128,), i32),
            pltpu.VMEM((CH,), i32),
            pltpu.VMEM((CH,), i32),
            pltpu.VMEM((CCAP,), i32),
            pltpu.VMEM((CCAP,), i32),
            pltpu.VMEM((WG,), i32),
            pltpu.VMEM((WG,), i32),
            pltpu.VMEM((WG,), i32),
            pltpu.VMEM((WG,), i32),
            pltpu.VMEM((WG, GROW), f32),
            pltpu.VMEM((WG, GROW), f32),
            pltpu.VMEM((WG * 16,), f32),
            pltpu.SemaphoreType.DMA,
            pltpu.SemaphoreType.DMA,
            pltpu.SemaphoreType.DMA,
            pltpu.SemaphoreType.DMA,
        ],
    )
    (p2,) = sc2(h2pk, bsrc, bdst, counts, ad2p, z2)

    return pl.pallas_call(
        _tc_c_body,
        grid=grid,
        in_specs=[
            pl.BlockSpec((blk, GROW), lambda i: (i, 0)),
            pl.BlockSpec((blk, GROW), lambda i: (i, 0)),
            pl.BlockSpec((1, NCO), lambda i: (0, 0)),
        ],
        out_specs=pl.BlockSpec((blk, NCO), lambda i: (i, 0)),
        out_shape=jax.ShapeDtypeStruct((N, NCO), f32),
    )(p2, h2pk, b2r)
